# baseline (speedup 1.0000x reference)
import functools

import jax
import jax.numpy as jnp
from jax.experimental import pallas as pl
from jax.experimental.pallas import tpu as pltpu

_IN = 3
_HID = 64
_OUT = 2


def _mlp_kernel(xt_ref, w1a_ref, w2_ref, b2_ref, o_ref):
    h = jnp.dot(w1a_ref[...], xt_ref[...],
                preferred_element_type=jnp.float32)
    hb = jnp.maximum(h.astype(jnp.bfloat16), jnp.bfloat16(0.0))
    y = jnp.dot(w2_ref[...], hb, preferred_element_type=jnp.float32)
    o_ref[...] = y + b2_ref[...]


@functools.partial(jax.jit, static_argnames=("tn",))
def _forward(x, w1, b1, w2, b2, *, tn=65536):
    n = x.shape[0]
    n_128 = max(128, ((n + 127) // 128) * 128)
    tile = min(tn, n_128)
    n_pad = ((n_128 + tile - 1) // tile) * tile
    grid = (n_pad // tile,)

    xt = jnp.concatenate(
        [jnp.pad(x.T, ((0, 0), (0, n_pad - n))),
         jnp.ones((1, n_pad), jnp.float32)], axis=0).astype(jnp.bfloat16)
    w1a = jnp.concatenate(
        [w1, b1.reshape(_HID, 1)], axis=1).astype(jnp.bfloat16)
    b2c = b2.reshape(_OUT, 1)

    out_t = pl.pallas_call(
        _mlp_kernel,
        out_shape=jax.ShapeDtypeStruct((_OUT, n_pad), jnp.float32),
        grid_spec=pl.GridSpec(
            grid=grid,
            in_specs=[
                pl.BlockSpec((_IN + 1, tile), lambda i: (0, i)),
                pl.BlockSpec((_HID, _IN + 1), lambda i: (0, 0)),
                pl.BlockSpec((_OUT, _HID), lambda i: (0, 0)),
                pl.BlockSpec((_OUT, 1), lambda i: (0, 0)),
            ],
            out_specs=pl.BlockSpec((_OUT, tile), lambda i: (0, i)),
        ),
        compiler_params=pltpu.CompilerParams(
            dimension_semantics=("parallel",),
        ),
    )(xt, w1a, w2.astype(jnp.bfloat16), b2c)

    return out_t[:, :n].T


def kernel(x, w1, b1, w2, b2):
    return _forward(x, w1, b1, w2, b2, tn=65536)


# speedup vs baseline: 1.2401x; 1.2401x over previous
import functools

import jax
import jax.numpy as jnp
from jax.experimental import pallas as pl
from jax.experimental.pallas import tpu as pltpu

_IN = 3
_HID = 64
_OUT = 2


def _mlp_kernel(xt_ref, w1_ref, b1_ref, w2_ref, b2_ref, o_ref):
    h = jnp.dot(w1_ref[...], xt_ref[...],
                preferred_element_type=jnp.float32)
    hb = h.astype(jnp.bfloat16) + b1_ref[...]
    hb = jnp.maximum(hb, jnp.bfloat16(0.0))
    y = jnp.dot(w2_ref[...], hb, preferred_element_type=jnp.float32)
    o_ref[...] = y + b2_ref[...]


@functools.partial(jax.jit, static_argnames=("tn",))
def _forward(x, w1, b1, w2, b2, *, tn=65536):
    n = x.shape[0]
    n_128 = max(128, ((n + 127) // 128) * 128)
    tile = min(tn, n_128)
    n_pad = ((n_128 + tile - 1) // tile) * tile
    grid = (n_pad // tile,)

    xt = jnp.pad(x.T, ((0, 0), (0, n_pad - n)))
    b1c = b1.reshape(_HID, 1).astype(jnp.bfloat16)
    b2c = b2.reshape(_OUT, 1)

    out_t = pl.pallas_call(
        _mlp_kernel,
        out_shape=jax.ShapeDtypeStruct((_OUT, n_pad), jnp.float32),
        grid_spec=pl.GridSpec(
            grid=grid,
            in_specs=[
                pl.BlockSpec((_IN, tile), lambda i: (0, i)),
                pl.BlockSpec((_HID, _IN), lambda i: (0, 0)),
                pl.BlockSpec((_HID, 1), lambda i: (0, 0)),
                pl.BlockSpec((_OUT, _HID), lambda i: (0, 0)),
                pl.BlockSpec((_OUT, 1), lambda i: (0, 0)),
            ],
            out_specs=pl.BlockSpec((_OUT, tile), lambda i: (0, i)),
        ),
        compiler_params=pltpu.CompilerParams(
            dimension_semantics=("parallel",),
        ),
    )(xt, w1, b1c, w2.astype(jnp.bfloat16), b2c)

    return out_t[:, :n].T


def kernel(x, w1, b1, w2, b2):
    return _forward(x, w1, b1, w2, b2, tn=65536)


# speedup vs baseline: 1.2691x; 1.0234x over previous
import functools

import jax
import jax.numpy as jnp
from jax.experimental import pallas as pl
from jax.experimental.pallas import tpu as pltpu

_IN = 3
_HID = 64
_OUT = 2


def _mlp_kernel(xt_ref, w1_ref, b1_ref, w2_ref, b2_ref, o_ref):
    h = jnp.dot(w1_ref[...], xt_ref[...],
                preferred_element_type=jnp.float32)
    hb = h.astype(jnp.bfloat16) + b1_ref[...]
    hb = jnp.maximum(hb, jnp.bfloat16(0.0))
    y = jnp.dot(w2_ref[...], hb, preferred_element_type=jnp.float32)
    o_ref[...] = y + b2_ref[...]


@functools.partial(jax.jit, static_argnames=("tn",))
def _forward(x, w1, b1, w2, b2, *, tn=131072):
    n = x.shape[0]
    n_128 = max(128, ((n + 127) // 128) * 128)
    tile = min(tn, n_128)
    n_pad = ((n_128 + tile - 1) // tile) * tile
    grid = (n_pad // tile,)

    xt = jnp.pad(x.T, ((0, 0), (0, n_pad - n)))
    b1c = b1.reshape(_HID, 1).astype(jnp.bfloat16)
    b2c = b2.reshape(_OUT, 1)

    out_t = pl.pallas_call(
        _mlp_kernel,
        out_shape=jax.ShapeDtypeStruct((_OUT, n_pad), jnp.float32),
        grid_spec=pl.GridSpec(
            grid=grid,
            in_specs=[
                pl.BlockSpec((_IN, tile), lambda i: (0, i)),
                pl.BlockSpec((_HID, _IN), lambda i: (0, 0)),
                pl.BlockSpec((_HID, 1), lambda i: (0, 0)),
                pl.BlockSpec((_OUT, _HID), lambda i: (0, 0)),
                pl.BlockSpec((_OUT, 1), lambda i: (0, 0)),
            ],
            out_specs=pl.BlockSpec((_OUT, tile), lambda i: (0, i)),
        ),
        compiler_params=pltpu.CompilerParams(
            dimension_semantics=("parallel",),
        ),
    )(xt, w1, b1c, w2.astype(jnp.bfloat16), b2c)

    return out_t[:, :n].T


def kernel(x, w1, b1, w2, b2):
    return _forward(x, w1, b1, w2, b2, tn=131072)
